# Initial kernel scaffold; baseline (speedup 1.0000x reference)
import jax
import jax.numpy as jnp
from jax.experimental import pallas as pl


def kernel(nuc_embedding, f_node_label, f_node_assignment, f_message, node_graph, message_graph, scope, all_dfs_idx, Wz_w, Wz_b, Wr_w, Ur_w, Ur_b, Wh_w, Wh_b, Out_w, Out_b, Wih_f, Whh_f, bih_f, bhh_f, Wih_b, Whh_b, bih_b, bhh_b):
    raise NotImplementedError("write your pallas kernel here")


# speedup vs baseline: 3.6398x; 3.6398x over previous
import functools

import jax
import jax.numpy as jnp
from jax import lax
from jax.experimental import pallas as pl
from jax.experimental.pallas import tpu as pltpu
from jax.experimental.pallas import tpu_sc as plsc

HID = 128
FDIM = 4
NC = 2
NS = 16
NW = NC * NS
CHUNK = 64

BN = 256

N_NODE = 75000
P_NODE = 75008
N_MSG = 100000
P_MSG = 100352
N_OUT = 50000
P_OUT = 50176


def _pad_cols(mat_t, p):
    k, n = mat_t.shape
    out = jnp.zeros((k, p), dtype=jnp.int32)
    out = lax.dynamic_update_slice(out, mat_t.astype(jnp.int32), (0, 0))
    return out.reshape(k * p)


def _sc_gather(table, idx_flat):
    b = idx_flat.shape[0]
    d = table.shape[1]
    b_per_w = b // NW
    n_chunks = b_per_w // CHUNK
    mesh = plsc.VectorSubcoreMesh(core_axis_name="c", subcore_axis_name="s")

    def body(table_hbm, idx_hbm, out_hbm, idx_v, rows_v, sem):
        wid = lax.axis_index("s") * NC + lax.axis_index("c")
        base = wid * b_per_w
        pltpu.sync_copy(idx_hbm.at[pl.ds(base, b_per_w)], idx_v)

        @pl.loop(0, n_chunks)
        def _(i):
            off = i * CHUNK
            pltpu.async_copy(
                table_hbm.at[idx_v.at[pl.ds(off, CHUNK)]], rows_v, sem
            ).wait()
            pltpu.sync_copy(rows_v, out_hbm.at[pl.ds(base + off, CHUNK)])

    f = pl.kernel(
        body,
        out_type=jax.ShapeDtypeStruct((b, d), jnp.float32),
        mesh=mesh,
        scratch_types=[
            pltpu.VMEM((b_per_w,), jnp.int32),
            pltpu.VMEM((CHUNK, d), jnp.float32),
            pltpu.SemaphoreType.DMA,
        ],
        compiler_params=pltpu.CompilerParams(use_tc_tiling_on_sc=(d % 128 == 0)),
    )
    return f(table, idx_flat)


def _rows(pid, n):
    return pid * BN + lax.broadcasted_iota(jnp.int32, (BN, n), 0)


def _k1_body(g_ref, idx_ref, out_ref):
    pid = pl.program_id(0)
    acc = jnp.where((idx_ref[0][:, None] >= 200000), 0.0, g_ref[0])
    for k in range(1, 8):
        v = jnp.where((idx_ref[k][:, None] >= 200000), 0.0, g_ref[k])
        acc = jnp.maximum(acc, v)
    acc = jnp.where(_rows(pid, HID) < N_NODE, acc, 0.0)
    out_ref[...] = acc


def _k1(g, asg_pad):
    return pl.pallas_call(
        _k1_body,
        grid=(P_NODE // BN,),
        in_specs=[
            pl.BlockSpec((8, BN, HID), lambda i: (0, i, 0)),
            pl.BlockSpec((8, BN), lambda i: (0, i)),
        ],
        out_specs=pl.BlockSpec((BN, HID), lambda i: (i, 0)),
        out_shape=jax.ShapeDtypeStruct((P_NODE, HID), jnp.float32),
    )(g, asg_pad)


def _k2_body(lab_ref, emb_ref, wlab_ref, wemb_ref, b_ref, l_ref, m_ref):
    pid = pl.program_id(0)
    labs = lab_ref[0] + lab_ref[1] + lab_ref[2] + lab_ref[3]
    embs = emb_ref[0] + emb_ref[1] + emb_ref[2] + emb_ref[3]
    l_all = (
        jnp.dot(labs, wlab_ref[...], preferred_element_type=jnp.float32)
        + jnp.dot(embs, wemb_ref[...], preferred_element_type=jnp.float32)
        + b_ref[...]
    )
    m1 = jax.nn.sigmoid(l_all[:, 0:HID]) * jnp.tanh(l_all[:, 2 * HID : 3 * HID])
    m1 = jnp.where(_rows(pid, HID) == 0, 0.0, m1)
    l_ref[...] = l_all
    m_ref[...] = m1


def _k2(lab, emb, wlab_t, wemb_t, b_all):
    return pl.pallas_call(
        _k2_body,
        grid=(P_MSG // BN,),
        in_specs=[
            pl.BlockSpec((4, BN, 16), lambda i: (0, i, 0)),
            pl.BlockSpec((4, BN, HID), lambda i: (0, i, 0)),
            pl.BlockSpec((16, 3 * HID), lambda i: (0, 0)),
            pl.BlockSpec((HID, 3 * HID), lambda i: (0, 0)),
            pl.BlockSpec((1, 3 * HID), lambda i: (0, 0)),
        ],
        out_specs=[
            pl.BlockSpec((BN, 3 * HID), lambda i: (i, 0)),
            pl.BlockSpec((BN, HID), lambda i: (i, 0)),
        ],
        out_shape=[
            jax.ShapeDtypeStruct((P_MSG, 3 * HID), jnp.float32),
            jax.ShapeDtypeStruct((P_MSG, HID), jnp.float32),
        ],
    )(lab, emb, wlab_t, wemb_t, b_all)


def _k3_body(m_ref, l_ref, wz_ref, ur_ref, wh_ref, urb_ref, out_ref):
    pid = pl.program_id(0)
    m0, m1, m2, m3 = m_ref[0], m_ref[1], m_ref[2], m_ref[3]
    summsg = m0 + m1 + m2 + m3
    l_all = l_ref[...]
    z = jax.nn.sigmoid(
        l_all[:, 0:HID]
        + jnp.dot(summsg, wz_ref[...], preferred_element_type=jnp.float32)
    )
    lrb = l_all[:, HID : 2 * HID] + urb_ref[...]
    sg = jnp.zeros_like(summsg)
    for mk in (m0, m1, m2, m3):
        r = jax.nn.sigmoid(
            lrb + jnp.dot(mk, ur_ref[...], preferred_element_type=jnp.float32)
        )
        sg = sg + r * mk
    pre = jnp.tanh(
        l_all[:, 2 * HID : 3 * HID]
        + jnp.dot(sg, wh_ref[...], preferred_element_type=jnp.float32)
    )
    out = (1.0 - z) * summsg + z * pre
    out = jnp.where(_rows(pid, HID) == 0, 0.0, out)
    out_ref[...] = out


def _k3(m, l_all, wz_t, ur_t, wh_t, ur_b):
    return pl.pallas_call(
        _k3_body,
        grid=(P_MSG // BN,),
        in_specs=[
            pl.BlockSpec((4, BN, HID), lambda i: (0, i, 0)),
            pl.BlockSpec((BN, 3 * HID), lambda i: (i, 0)),
            pl.BlockSpec((HID, HID), lambda i: (0, 0)),
            pl.BlockSpec((HID, HID), lambda i: (0, 0)),
            pl.BlockSpec((HID, HID), lambda i: (0, 0)),
            pl.BlockSpec((1, HID), lambda i: (0, 0)),
        ],
        out_specs=pl.BlockSpec((BN, HID), lambda i: (i, 0)),
        out_shape=jax.ShapeDtypeStruct((P_MSG, HID), jnp.float32),
    )(m, l_all, wz_t, ur_t, wh_t, ur_b)


def _k4_body(lab_ref, emb_ref, inc_ref, wl_ref, we_ref, wi_ref, b_ref, out_ref):
    labs = lab_ref[0] + lab_ref[1] + lab_ref[2] + lab_ref[3]
    embs = emb_ref[0] + emb_ref[1] + emb_ref[2] + emb_ref[3]
    incs = inc_ref[0] + inc_ref[1] + inc_ref[2] + inc_ref[3]
    h = (
        jnp.dot(labs, wl_ref[...], preferred_element_type=jnp.float32)
        + jnp.dot(embs, we_ref[...], preferred_element_type=jnp.float32)
        + jnp.dot(incs, wi_ref[...], preferred_element_type=jnp.float32)
        + b_ref[...]
    )
    out_ref[...] = jnp.maximum(h, 0.0)


def _k4(lab, emb, inc, wl_t, we_t, wi_t, b):
    return pl.pallas_call(
        _k4_body,
        grid=(P_OUT // BN,),
        in_specs=[
            pl.BlockSpec((4, BN, 16), lambda i: (0, i, 0)),
            pl.BlockSpec((4, BN, HID), lambda i: (0, i, 0)),
            pl.BlockSpec((4, BN, HID), lambda i: (0, i, 0)),
            pl.BlockSpec((16, HID), lambda i: (0, 0)),
            pl.BlockSpec((HID, HID), lambda i: (0, 0)),
            pl.BlockSpec((HID, HID), lambda i: (0, 0)),
            pl.BlockSpec((1, HID), lambda i: (0, 0)),
        ],
        out_specs=pl.BlockSpec((BN, HID), lambda i: (i, 0)),
        out_shape=jax.ShapeDtypeStruct((P_OUT, HID), jnp.float32),
    )(lab, emb, inc, wl_t, we_t, wi_t, b)


def _k5_body(x_ref, w_ref, b_ref, out_ref):
    out_ref[...] = (
        jnp.dot(x_ref[...], w_ref[...], preferred_element_type=jnp.float32)
        + b_ref[...]
    )


def _k5(seq, w_t, b):
    n = seq.shape[0]
    return pl.pallas_call(
        _k5_body,
        grid=(n // BN,),
        in_specs=[
            pl.BlockSpec((BN, HID), lambda i: (i, 0)),
            pl.BlockSpec((HID, 512), lambda i: (0, 0)),
            pl.BlockSpec((1, 512), lambda i: (0, 0)),
        ],
        out_specs=pl.BlockSpec((BN, 512), lambda i: (i, 0)),
        out_shape=jax.ShapeDtypeStruct((n, 512), jnp.float32),
    )(seq, w_t, b)


H2 = HID // 2
T_STEPS = 512
BATCH = 64


def _lstm_step(g, h, c, whh_ref):
    g = g + jnp.dot(h, whh_ref[...], preferred_element_type=jnp.float32)
    i = jax.nn.sigmoid(g[:, 0:H2])
    f = jax.nn.sigmoid(g[:, H2 : 2 * H2])
    gg = jnp.tanh(g[:, 2 * H2 : 3 * H2])
    o = jax.nn.sigmoid(g[:, 3 * H2 : 4 * H2])
    c_new = f * c + i * gg
    h_new = o * jnp.tanh(c_new)
    return h_new, c_new


def _k6_body(gf_ref, gb_ref, whf_ref, whb_ref, out_ref, hf, cf, hb, cb, mf, mb):
    t = pl.program_id(0)

    @pl.when(t == 0)
    def _():
        z = jnp.zeros((BATCH, H2), jnp.float32)
        hf[...] = z
        cf[...] = z
        hb[...] = z
        cb[...] = z
        mf[...] = jnp.full((BATCH, H2), -jnp.inf, jnp.float32)
        mb[...] = jnp.full((BATCH, H2), -jnp.inf, jnp.float32)

    h_new, c_new = _lstm_step(gf_ref[0], hf[...], cf[...], whf_ref)
    hf[...] = h_new
    cf[...] = c_new
    mf[...] = jnp.maximum(mf[...], h_new)

    h_new, c_new = _lstm_step(gb_ref[0], hb[...], cb[...], whb_ref)
    hb[...] = h_new
    cb[...] = c_new
    mb[...] = jnp.maximum(mb[...], h_new)

    @pl.when(t == T_STEPS - 1)
    def _():
        out_ref[...] = jnp.concatenate([mf[...], mb[...]], axis=1)


def _k6(gates3, whf_t, whb_t):
    return pl.pallas_call(
        _k6_body,
        grid=(T_STEPS,),
        in_specs=[
            pl.BlockSpec((1, BATCH, 4 * H2), lambda t: (t, 0, 0)),
            pl.BlockSpec((1, BATCH, 4 * H2), lambda t: (T_STEPS - 1 - t, 0, 1)),
            pl.BlockSpec((H2, 4 * H2), lambda t: (0, 0)),
            pl.BlockSpec((H2, 4 * H2), lambda t: (0, 0)),
        ],
        out_specs=pl.BlockSpec((BATCH, HID), lambda t: (0, 0)),
        out_shape=jax.ShapeDtypeStruct((BATCH, HID), jnp.float32),
        scratch_shapes=[pltpu.VMEM((BATCH, H2), jnp.float32)] * 6,
    )(gates3, gates3, whf_t, whb_t)


def kernel(nuc_embedding, f_node_label, f_node_assignment, f_message, node_graph,
           message_graph, scope, all_dfs_idx,
           Wz_w, Wz_b, Wr_w, Ur_w, Ur_b, Wh_w, Wh_b, Out_w, Out_b,
           Wih_f, Whh_f, bih_f, bhh_f, Wih_b, Whh_b, bih_b, bhh_b):
    f32 = jnp.float32

    wlab_t = jnp.zeros((16, 3 * HID), f32)
    wlab = jnp.concatenate(
        [Wz_w[:, 0:FDIM], Wr_w[:, 0:FDIM], Wh_w[:, 0:FDIM]], axis=0
    )
    wlab_t = lax.dynamic_update_slice(wlab_t, wlab.T, (0, 0))
    wemb_t = jnp.concatenate(
        [Wz_w[:, FDIM : FDIM + HID], Wr_w[:, FDIM:], Wh_w[:, FDIM : FDIM + HID]],
        axis=0,
    ).T
    b_all = jnp.concatenate([Wz_b, jnp.zeros((HID,), f32), Wh_b]).reshape(1, 3 * HID)
    wz2_t = Wz_w[:, FDIM + HID :].T
    wh2_t = Wh_w[:, FDIM + HID :].T
    ur_t = Ur_w.T
    ur_b = Ur_b.reshape(1, HID)

    olab_t = jnp.zeros((16, HID), f32)
    olab_t = lax.dynamic_update_slice(olab_t, Out_w[:, 0:FDIM].T, (0, 0))
    oemb_t = Out_w[:, FDIM : FDIM + HID].T
    oinc_t = Out_w[:, FDIM + HID :].T
    out_b = Out_b.reshape(1, HID)

    wih_t = jnp.concatenate([Wih_f, Wih_b], axis=0).T
    bcat = jnp.concatenate([bih_f + bhh_f, bih_b + bhh_b]).reshape(1, 512)
    whf_t = Whh_f.T
    whb_t = Whh_b.T

    asg = f_node_assignment.astype(jnp.int32).T
    asg_pad = jnp.zeros((8, P_NODE), jnp.int32)
    asg_pad = lax.dynamic_update_slice(asg_pad, asg, (0, 0))
    nuc_idx = jnp.minimum(asg_pad, 200000 - 1).reshape(8 * P_NODE)

    fmsg_idx = _pad_cols(f_message.T, P_MSG)
    mg_idx = _pad_cols(message_graph.T, P_MSG)
    ng_idx = _pad_cols(node_graph.T, P_OUT)
    dfs_idx = all_dfs_idx.astype(jnp.int32).T.reshape(T_STEPS * BATCH)

    lab_tab = jnp.zeros((P_NODE, 16), f32)
    lab_tab = lax.dynamic_update_slice(lab_tab, f_node_label, (0, 0))

    g_nuc = _sc_gather(nuc_embedding, nuc_idx).reshape(8, P_NODE, HID)
    node_emb = _k1(g_nuc, asg_pad)

    loc_lab = _sc_gather(lab_tab, fmsg_idx).reshape(4, P_MSG, 16)
    loc_emb = _sc_gather(node_emb, fmsg_idx).reshape(4, P_MSG, HID)
    l_all, messages = _k2(loc_lab, loc_emb, wlab_t, wemb_t, b_all)

    for _ in range(1):
        m_nei = _sc_gather(messages, mg_idx).reshape(4, P_MSG, HID)
        messages = _k3(m_nei, l_all, wz2_t, ur_t, wh2_t, ur_b)

    fn2_lab = _sc_gather(lab_tab, ng_idx).reshape(4, P_OUT, 16)
    fn2_emb = _sc_gather(node_emb, ng_idx).reshape(4, P_OUT, HID)
    inc = _sc_gather(messages, ng_idx).reshape(4, P_OUT, HID)
    hpn = _k4(fn2_lab, fn2_emb, inc, olab_t, oemb_t, oinc_t, out_b)

    seq = _sc_gather(hpn, dfs_idx)
    gates = _k5(seq, wih_t, bcat)
    gates3 = gates.reshape(T_STEPS, BATCH, 512)
    return _k6(gates3, whf_t, whb_t)
